# Initial kernel scaffold; baseline (speedup 1.0000x reference)
import jax
import jax.numpy as jnp
from jax.experimental import pallas as pl


def kernel(x, edge_index, Wl0, bl0, Wr0, g0, be0, Wl1, bl1, Wr1, g1, be1, Wl2, bl2, Wr2, g2, be2, Wc1, bc1, Wc2, bc2):
    raise NotImplementedError("write your pallas kernel here")


# speedup vs baseline: 7.1460x; 7.1460x over previous
import jax
import jax.numpy as jnp
from jax import lax
from jax.experimental import pallas as pl
from jax.experimental.pallas import tpu as pltpu
from jax.experimental.pallas import tpu_sc as plsc

_N = 10000
_E = 320000
_D = 128
_NC = 2
_NS = 16
_NW = _NC * _NS
_EPW = _E // _NW
_CH = 125
_NCH = _EPW // _CH
_IG = 16
_NG = _NCH // _IG
_RPT = 624
_REM = _N - _NS * _RPT
_ZR = 48
_L = 16


def _sc_mesh():
    return plsc.VectorSubcoreMesh(core_axis_name="c", subcore_axis_name="s")


def _make_edge_pass():
    out_type = (jax.ShapeDtypeStruct((_NC, _N, _D), jnp.float32),)
    scratch = (
        pltpu.VMEM((_IG, _CH), jnp.int32),
        pltpu.VMEM((_IG, _CH), jnp.int32),
        pltpu.VMEM((_CH, _D), jnp.float32),
        pltpu.VMEM((_ZR, _D), jnp.float32),
        pltpu.VMEM_SHARED((_N, _D), jnp.float32),
        pltpu.SemaphoreType.DMA,
    )

    def body(y_hbm, src_hbm, dst_hbm, agg_hbm, src_v, dst_v, rows_v,
             zbuf, acc, sem):
        cid = lax.axis_index("c")
        sid = lax.axis_index("s")
        wid = cid * _NS + sid

        base = pl.multiple_of(sid * _RPT, 8)

        def zrow(i, _):
            for j in range(_D // _L):
                zbuf[i, pl.ds(j * _L, _L)] = jnp.zeros((_L,), jnp.float32)
            return 0

        lax.fori_loop(0, _ZR, zrow, 0)
        for k in range(_RPT // _ZR):
            pltpu.sync_copy(zbuf, acc.at[pl.ds(base + k * _ZR, _ZR)])

        @pl.when(sid == _NS - 1)
        def _zero_rem():
            pltpu.sync_copy(zbuf.at[pl.ds(0, _REM)],
                            acc.at[pl.ds(_NS * _RPT, _REM)])

        plsc.subcore_barrier()

        for g in range(_NG):
            gsl = pl.ds(g * _IG, _IG)
            pltpu.sync_copy(src_hbm.at[wid, gsl], src_v)
            pltpu.sync_copy(dst_hbm.at[wid, gsl], dst_v)

            def chunk(j, _):
                pltpu.async_copy(y_hbm.at[src_v.at[j]], rows_v, sem).wait()
                pltpu.sync_copy(rows_v, acc.at[dst_v.at[j]], add=True)
                return 0

            lax.fori_loop(0, _IG, chunk, 0)

        plsc.subcore_barrier()

        for k in range(_RPT // _ZR):
            sl = pl.ds(base + k * _ZR, _ZR)
            pltpu.sync_copy(acc.at[sl], agg_hbm.at[cid, sl])

        @pl.when(sid == _NS - 1)
        def _out_rem():
            sl = pl.ds(_NS * _RPT, _REM)
            pltpu.sync_copy(acc.at[sl], agg_hbm.at[cid, sl])

    return pl.kernel(body, out_type=out_type, mesh=_sc_mesh(),
                     scratch_types=scratch)


def _make_cnt_pass():
    out_type = (jax.ShapeDtypeStruct((_NC, _N, _D), jnp.float32),)
    scratch = (
        pltpu.VMEM((_IG, _CH), jnp.int32),
        pltpu.VMEM((_CH, _D), jnp.float32),
        pltpu.VMEM((_ZR, _D), jnp.float32),
        pltpu.VMEM_SHARED((_N, _D), jnp.float32),
    )

    def body(dst_hbm, cnt_hbm, dst_v, ones_v, zbuf, cacc):
        cid = lax.axis_index("c")
        sid = lax.axis_index("s")
        wid = cid * _NS + sid

        base = pl.multiple_of(sid * _RPT, 8)

        def orow(i, _):
            for j in range(_D // _L):
                ones_v[i, pl.ds(j * _L, _L)] = jnp.ones((_L,), jnp.float32)
            return 0

        lax.fori_loop(0, _CH, orow, 0)

        def zrow(i, _):
            for j in range(_D // _L):
                zbuf[i, pl.ds(j * _L, _L)] = jnp.zeros((_L,), jnp.float32)
            return 0

        lax.fori_loop(0, _ZR, zrow, 0)
        for k in range(_RPT // _ZR):
            pltpu.sync_copy(zbuf, cacc.at[pl.ds(base + k * _ZR, _ZR)])

        @pl.when(sid == _NS - 1)
        def _zero_rem():
            pltpu.sync_copy(zbuf.at[pl.ds(0, _REM)],
                            cacc.at[pl.ds(_NS * _RPT, _REM)])

        plsc.subcore_barrier()

        for g in range(_NG):
            pltpu.sync_copy(dst_hbm.at[wid, pl.ds(g * _IG, _IG)], dst_v)

            def chunk(j, _):
                pltpu.sync_copy(ones_v, cacc.at[dst_v.at[j]], add=True)
                return 0

            lax.fori_loop(0, _IG, chunk, 0)

        plsc.subcore_barrier()

        for k in range(_RPT // _ZR):
            sl = pl.ds(base + k * _ZR, _ZR)
            pltpu.sync_copy(cacc.at[sl], cnt_hbm.at[cid, sl])

        @pl.when(sid == _NS - 1)
        def _out_rem():
            sl = pl.ds(_NS * _RPT, _REM)
            pltpu.sync_copy(cacc.at[sl], cnt_hbm.at[cid, sl])

    return pl.kernel(body, out_type=out_type, mesh=_sc_mesh(),
                     scratch_types=scratch)


_edge_pass = _make_edge_pass()
_cnt_pass = _make_cnt_pass()


_BN = 1000
_GN = _N // _BN


def _dot_t(a, w):
    return lax.dot_general(a, w, (((1,), (1,)), ((), ())),
                           preferred_element_type=jnp.float32)


def _full(shape):
    return pl.BlockSpec(shape, lambda i: (0,) * len(shape))


def _rows(w):
    return pl.BlockSpec((_BN, w), lambda i: (i, 0))


def _tc_pre(x, Wl, Wr, bl):
    def bdy(x_ref, wl_ref, wr_ref, bl_ref, y_ref, r_ref):
        xb = x_ref[...]
        y_ref[...] = _dot_t(xb, wl_ref[...])
        r_ref[...] = _dot_t(xb, wr_ref[...]) + bl_ref[...]

    return pl.pallas_call(
        bdy,
        grid=(_GN,),
        in_specs=[_rows(_D), _full((_D, _D)), _full((_D, _D)),
                  _full((1, _D))],
        out_specs=[_rows(_D), _rows(_D)],
        out_shape=[jax.ShapeDtypeStruct((_N, _D), jnp.float32)] * 2,
    )(x, Wl, Wr, bl.reshape(1, _D))


def _combine(a0, a1, c0, c1, r, g, be):
    cnt = c0[:, 0:1] + c1[:, 0:1]
    inv = 1.0 / jnp.maximum(cnt, 1.0)
    t = (a0 + a1) * inv + r
    mu = jnp.mean(t, axis=-1, keepdims=True)
    var = jnp.mean((t - mu) ** 2, axis=-1, keepdims=True)
    t = (t - mu) * lax.rsqrt(var + 1e-5) * g + be
    return jnp.maximum(t, 0.0)


def _tc_mid(residual):
    def bdy(a0_ref, a1_ref, c0_ref, c1_ref, r_ref, *rest):
        if residual:
            (h_ref, g_ref, be_ref, wl_ref, wr_ref, bl_ref,
             h_out, y_out, r_out) = rest
        else:
            (g_ref, be_ref, wl_ref, wr_ref, bl_ref,
             h_out, y_out, r_out) = rest
        t = _combine(a0_ref[...], a1_ref[...], c0_ref[...], c1_ref[...],
                     r_ref[...], g_ref[...], be_ref[...])
        if residual:
            t = t + h_ref[...]
        h_out[...] = t
        y_out[...] = _dot_t(t, wl_ref[...])
        r_out[...] = _dot_t(t, wr_ref[...]) + bl_ref[...]

    n_h = [_rows(_D)] if residual else []
    call = pl.pallas_call(
        bdy,
        grid=(_GN,),
        in_specs=([_rows(_D), _rows(_D), _rows(_D), _rows(_D), _rows(_D)]
                  + n_h
                  + [_full((1, _D)), _full((1, _D)), _full((_D, _D)),
                     _full((_D, _D)), _full((1, _D))]),
        out_specs=[_rows(_D)] * 3,
        out_shape=[jax.ShapeDtypeStruct((_N, _D), jnp.float32)] * 3,
    )

    def run(a0, a1, c0, c1, r, h, g, be, Wl, Wr, bl):
        args = [a0, a1, c0, c1, r] + ([h] if residual else []) + [
            g.reshape(1, _D), be.reshape(1, _D), Wl, Wr, bl.reshape(1, _D)]
        return call(*args)

    return run


_tc_mid0 = _tc_mid(False)
_tc_mid1 = _tc_mid(True)


def _tc_final(a0, a1, c0, c1, r, h, g, be, Wc1, bc1, Wc2, bc2):
    def bdy(a0_ref, a1_ref, c0_ref, c1_ref, r_ref, h_ref, g_ref, be_ref,
            wc1_ref, bc1_ref, wc2_ref, bc2_ref, out_ref):
        t = _combine(a0_ref[...], a1_ref[...], c0_ref[...], c1_ref[...],
                     r_ref[...], g_ref[...], be_ref[...])
        t = t + h_ref[...]
        z = jnp.maximum(_dot_t(t, wc1_ref[...]) + bc1_ref[...], 0.0)
        out_ref[...] = _dot_t(z, wc2_ref[...]) + bc2_ref[0]

    Wc2p = jnp.pad(Wc2, ((0, 7), (0, 0)))

    return pl.pallas_call(
        bdy,
        grid=(_GN,),
        in_specs=[_rows(_D), _rows(_D), _rows(_D), _rows(_D), _rows(_D),
                  _rows(_D), _full((1, _D)), _full((1, _D)),
                  _full((_D // 2, _D)), _full((1, _D // 2)),
                  _full((8, _D // 2)),
                  pl.BlockSpec(memory_space=pltpu.SMEM)],
        out_specs=[pl.BlockSpec((_BN, 8), lambda i: (i, 0))],
        out_shape=[jax.ShapeDtypeStruct((_N, 8), jnp.float32)],
    )(a0, a1, c0, c1, r, h, g.reshape(1, _D), be.reshape(1, _D),
      Wc1, bc1.reshape(1, _D // 2), Wc2p, bc2)[0]


def kernel(x, edge_index, Wl0, bl0, Wr0, g0, be0, Wl1, bl1, Wr1, g1, be1,
           Wl2, bl2, Wr2, g2, be2, Wc1, bc1, Wc2, bc2):
    src3 = edge_index[0].reshape(_NW, _NCH, _CH)
    dst3 = edge_index[1].reshape(_NW, _NCH, _CH)

    _DBG = False
    _DBG_CNT = False
    if _DBG:
        src1, dst1 = edge_index[0], edge_index[1]

        def _sseg(y):
            a = jax.ops.segment_sum(jnp.take(y, src1, axis=0), dst1,
                                    num_segments=_N)
            return jnp.stack([a, jnp.zeros_like(a)])

        if _DBG_CNT:
            cnt_flat = jax.ops.segment_sum(
                jnp.ones_like(src1, jnp.float32), dst1, num_segments=_N)
            cnt = jnp.stack([jnp.broadcast_to(cnt_flat[:, None], (_N, _L)),
                             jnp.zeros((_N, _L), jnp.float32)])
        else:
            (cnt,) = _cnt_pass(dst3)

    y0, r0 = _tc_pre(x, Wl0, Wr0, bl0)
    if not _DBG:
        (cnt,) = _cnt_pass(dst3)
        (agg0,) = _edge_pass(y0, src3, dst3)
    else:
        agg0 = _sseg(y0)
    c0, c1 = cnt[0], cnt[1]
    h1, y1, r1 = _tc_mid0(agg0[0], agg0[1], c0, c1, r0, None,
                          g0, be0, Wl1, Wr1, bl1)
    (agg1,) = (_sseg(y1),) if _DBG else _edge_pass(y1, src3, dst3)
    h2, y2, r2 = _tc_mid1(agg1[0], agg1[1], c0, c1, r1, h1,
                          g1, be1, Wl2, Wr2, bl2)
    (agg2,) = (_sseg(y2),) if _DBG else _edge_pass(y2, src3, dst3)
    out = _tc_final(agg2[0], agg2[1], c0, c1, r2, h2,
                    g2, be2, Wc1, bc1, Wc2, bc2)
    return out[:, 0]


# speedup vs baseline: 8.0283x; 1.1235x over previous
import jax
import jax.numpy as jnp
from jax import lax
from jax.experimental import pallas as pl
from jax.experimental.pallas import tpu as pltpu
from jax.experimental.pallas import tpu_sc as plsc

_N = 10000
_E = 320000
_D = 128
_NC = 2
_NS = 16
_NW = _NC * _NS
_EPW = _E // _NW
_CH = 100
_IG = 10
_NG = _EPW // (_CH * _IG)
_RPT = 624
_REM = _N - _NS * _RPT
_L = 16


def _sc_mesh():
    return plsc.VectorSubcoreMesh(core_axis_name="c", subcore_axis_name="s")


def _zero_fill(buf, nrows):
    def zrow(i, _):
        for j in range(_D // _L):
            buf[i, pl.ds(j * _L, _L)] = jnp.zeros((_L,), jnp.float32)
        return 0

    lax.fori_loop(0, nrows, zrow, 0)


def _zero_acc(acc, zsrc, base):
    for k in range(6):
        pltpu.sync_copy(zsrc, acc.at[pl.ds(base + k * 96, 96)])
    pltpu.sync_copy(zsrc.at[pl.ds(0, 48)], acc.at[pl.ds(base + 576, 48)])


def _write_out(src_ref, out_hbm, cid, sid, base):
    sl = pl.ds(base, _RPT)
    pltpu.sync_copy(src_ref.at[sl], out_hbm.at[cid, sl])

    @pl.when(sid == _NS - 1)
    def _rem():
        sl2 = pl.ds(_NS * _RPT, _REM)
        pltpu.sync_copy(src_ref.at[sl2], out_hbm.at[cid, sl2])


def _make_edge_pass():
    out_type = (jax.ShapeDtypeStruct((_NC, _N, _D), jnp.float32),)
    scratch = (
        pltpu.VMEM((_IG, _CH), jnp.int32),
        pltpu.VMEM((_IG, _CH), jnp.int32),
        pltpu.VMEM((_CH, _D), jnp.float32),
        pltpu.VMEM((_CH, _D), jnp.float32),
        pltpu.VMEM_SHARED((_N, _D), jnp.float32),
        pltpu.SemaphoreType.DMA,
        pltpu.SemaphoreType.DMA,
        pltpu.SemaphoreType.DMA,
    )

    def body(y_hbm, src_hbm, dst_hbm, agg_hbm, src_v, dst_v, rows_a, rows_b,
             acc, semg, sema, semb):
        cid = lax.axis_index("c")
        sid = lax.axis_index("s")
        wid = cid * _NS + sid
        base = pl.multiple_of(sid * _RPT, 8)

        _zero_fill(rows_a, 96)
        _zero_acc(acc, rows_a.at[pl.ds(0, 96)], base)

        @pl.when(sid == _NS - 1)
        def _zrem():
            pltpu.sync_copy(rows_a.at[pl.ds(0, _REM)],
                            acc.at[pl.ds(_NS * _RPT, _REM)])

        plsc.subcore_barrier()

        for g in range(_NG):
            pltpu.sync_copy(src_hbm.at[wid, g], src_v)
            pltpu.sync_copy(dst_hbm.at[wid, g], dst_v)
            pltpu.async_copy(y_hbm.at[src_v.at[0]], rows_a, semg)

            def pair(k, _):
                j = 2 * k
                pltpu.make_async_copy(y_hbm.at[src_v.at[j]], rows_a,
                                      semg).wait()
                sca = pltpu.async_copy(rows_a, acc.at[dst_v.at[j]], sema,
                                       add=True)
                pltpu.async_copy(y_hbm.at[src_v.at[j + 1]], rows_b, semg)
                pltpu.make_async_copy(y_hbm.at[src_v.at[j + 1]], rows_b,
                                      semg).wait()
                scb = pltpu.async_copy(rows_b, acc.at[dst_v.at[j + 1]], semb,
                                       add=True)
                sca.wait()

                @pl.when(k < _IG // 2 - 1)
                def _prefetch():
                    pltpu.async_copy(y_hbm.at[src_v.at[j + 2]], rows_a, semg)

                scb.wait()
                return 0

            lax.fori_loop(0, _IG // 2, pair, 0)

        plsc.subcore_barrier()
        _write_out(acc, agg_hbm, cid, sid, base)

    return pl.kernel(body, out_type=out_type, mesh=_sc_mesh(),
                     scratch_types=scratch)


def _make_cnt_pass():
    out_type = (jax.ShapeDtypeStruct((_NC, _N, _D), jnp.float32),)
    scratch = (
        pltpu.VMEM((_IG, _CH), jnp.int32),
        pltpu.VMEM((_CH, _D), jnp.float32),
        pltpu.VMEM_SHARED((_N, _D), jnp.float32),
        pltpu.SemaphoreType.DMA,
    )

    def body(dst_hbm, cnt_hbm, dst_v, ones_v, cacc, sem):
        cid = lax.axis_index("c")
        sid = lax.axis_index("s")
        wid = cid * _NS + sid
        base = pl.multiple_of(sid * _RPT, 8)

        _zero_fill(ones_v, 96)
        _zero_acc(cacc, ones_v.at[pl.ds(0, 96)], base)

        @pl.when(sid == _NS - 1)
        def _zrem():
            pltpu.sync_copy(ones_v.at[pl.ds(0, _REM)],
                            cacc.at[pl.ds(_NS * _RPT, _REM)])

        def orow(i, _):
            for j in range(_D // _L):
                ones_v[i, pl.ds(j * _L, _L)] = jnp.ones((_L,), jnp.float32)

            return 0

        lax.fori_loop(0, _CH, orow, 0)

        plsc.subcore_barrier()

        for g in range(_NG):
            pltpu.sync_copy(dst_hbm.at[wid, g], dst_v)

            def sc1(j, _):
                pltpu.async_copy(ones_v, cacc.at[dst_v.at[j]], sem, add=True)

                @pl.when(j >= 4)
                def _drain_one():
                    pltpu.make_async_copy(
                        ones_v, cacc.at[dst_v.at[0]], sem).wait()

                return 0

            lax.fori_loop(0, _IG, sc1, 0)
            for _ in range(4):
                pltpu.make_async_copy(ones_v, cacc.at[dst_v.at[0]],
                                      sem).wait()

        plsc.subcore_barrier()
        _write_out(cacc, cnt_hbm, cid, sid, base)

    return pl.kernel(body, out_type=out_type, mesh=_sc_mesh(),
                     scratch_types=scratch)


_edge_pass = _make_edge_pass()
_cnt_pass = _make_cnt_pass()


_BN = 1000
_GN = _N // _BN


def _dot_t(a, w):
    return lax.dot_general(a, w, (((1,), (1,)), ((), ())),
                           preferred_element_type=jnp.float32)


def _full(shape):
    return pl.BlockSpec(shape, lambda i: (0,) * len(shape))


def _rows(w):
    return pl.BlockSpec((_BN, w), lambda i: (i, 0))


def _tc_pre(x, Wl, Wr, bl):
    def bdy(x_ref, wl_ref, wr_ref, bl_ref, y_ref, r_ref):
        xb = x_ref[...]
        y_ref[...] = _dot_t(xb, wl_ref[...])
        r_ref[...] = _dot_t(xb, wr_ref[...]) + bl_ref[...]

    return pl.pallas_call(
        bdy,
        grid=(_GN,),
        in_specs=[_rows(_D), _full((_D, _D)), _full((_D, _D)),
                  _full((1, _D))],
        out_specs=[_rows(_D), _rows(_D)],
        out_shape=[jax.ShapeDtypeStruct((_N, _D), jnp.float32)] * 2,
    )(x, Wl, Wr, bl.reshape(1, _D))


def _combine(a0, a1, c0, c1, r, g, be):
    cnt = c0[:, 0:1] + c1[:, 0:1]
    inv = 1.0 / jnp.maximum(cnt, 1.0)
    t = (a0 + a1) * inv + r
    mu = jnp.mean(t, axis=-1, keepdims=True)
    var = jnp.mean((t - mu) ** 2, axis=-1, keepdims=True)
    t = (t - mu) * lax.rsqrt(var + 1e-5) * g + be
    return jnp.maximum(t, 0.0)


def _tc_mid(residual):
    def bdy(a0_ref, a1_ref, c0_ref, c1_ref, r_ref, *rest):
        if residual:
            (h_ref, g_ref, be_ref, wl_ref, wr_ref, bl_ref,
             h_out, y_out, r_out) = rest
        else:
            (g_ref, be_ref, wl_ref, wr_ref, bl_ref,
             h_out, y_out, r_out) = rest
        t = _combine(a0_ref[...], a1_ref[...], c0_ref[...], c1_ref[...],
                     r_ref[...], g_ref[...], be_ref[...])
        if residual:
            t = t + h_ref[...]
        h_out[...] = t
        y_out[...] = _dot_t(t, wl_ref[...])
        r_out[...] = _dot_t(t, wr_ref[...]) + bl_ref[...]

    n_h = [_rows(_D)] if residual else []
    call = pl.pallas_call(
        bdy,
        grid=(_GN,),
        in_specs=([_rows(_D), _rows(_D), _rows(_D), _rows(_D), _rows(_D)]
                  + n_h
                  + [_full((1, _D)), _full((1, _D)), _full((_D, _D)),
                     _full((_D, _D)), _full((1, _D))]),
        out_specs=[_rows(_D)] * 3,
        out_shape=[jax.ShapeDtypeStruct((_N, _D), jnp.float32)] * 3,
    )

    def run(a0, a1, c0, c1, r, h, g, be, Wl, Wr, bl):
        args = [a0, a1, c0, c1, r] + ([h] if residual else []) + [
            g.reshape(1, _D), be.reshape(1, _D), Wl, Wr, bl.reshape(1, _D)]
        return call(*args)

    return run


_tc_mid0 = _tc_mid(False)
_tc_mid1 = _tc_mid(True)


def _tc_final(a0, a1, c0, c1, r, h, g, be, Wc1, bc1, Wc2, bc2):
    def bdy(a0_ref, a1_ref, c0_ref, c1_ref, r_ref, h_ref, g_ref, be_ref,
            wc1_ref, bc1_ref, wc2_ref, bc2_ref, out_ref):
        t = _combine(a0_ref[...], a1_ref[...], c0_ref[...], c1_ref[...],
                     r_ref[...], g_ref[...], be_ref[...])
        t = t + h_ref[...]
        z = jnp.maximum(_dot_t(t, wc1_ref[...]) + bc1_ref[...], 0.0)
        out_ref[...] = _dot_t(z, wc2_ref[...]) + bc2_ref[0]

    Wc2p = jnp.pad(Wc2, ((0, 7), (0, 0)))

    return pl.pallas_call(
        bdy,
        grid=(_GN,),
        in_specs=[_rows(_D), _rows(_D), _rows(_D), _rows(_D), _rows(_D),
                  _rows(_D), _full((1, _D)), _full((1, _D)),
                  _full((_D // 2, _D)), _full((1, _D // 2)),
                  _full((8, _D // 2)),
                  pl.BlockSpec(memory_space=pltpu.SMEM)],
        out_specs=[pl.BlockSpec((_BN, 8), lambda i: (i, 0))],
        out_shape=[jax.ShapeDtypeStruct((_N, 8), jnp.float32)],
    )(a0, a1, c0, c1, r, h, g.reshape(1, _D), be.reshape(1, _D),
      Wc1, bc1.reshape(1, _D // 2), Wc2p, bc2)[0]


def kernel(x, edge_index, Wl0, bl0, Wr0, g0, be0, Wl1, bl1, Wr1, g1, be1,
           Wl2, bl2, Wr2, g2, be2, Wc1, bc1, Wc2, bc2):
    src4 = edge_index[0].reshape(_NW, _NG, _IG, _CH)
    dst4 = edge_index[1].reshape(_NW, _NG, _IG, _CH)

    y0, r0 = _tc_pre(x, Wl0, Wr0, bl0)
    (cnt,) = _cnt_pass(dst4)
    (agg0,) = _edge_pass(y0, src4, dst4)
    c0, c1 = cnt[0], cnt[1]
    h1, y1, r1 = _tc_mid0(agg0[0], agg0[1], c0, c1, r0, None,
                          g0, be0, Wl1, Wr1, bl1)
    (agg1,) = _edge_pass(y1, src4, dst4)
    h2, y2, r2 = _tc_mid1(agg1[0], agg1[1], c0, c1, r1, h1,
                          g1, be1, Wl2, Wr2, bl2)
    (agg2,) = _edge_pass(y2, src4, dst4)
    out = _tc_final(agg2[0], agg2[1], c0, c1, r2, h2,
                    g2, be2, Wc1, bc1, Wc2, bc2)
    return out[:, 0]


# speedup vs baseline: 8.8058x; 1.0968x over previous
import jax
import jax.numpy as jnp
from jax import lax
from jax.experimental import pallas as pl
from jax.experimental.pallas import tpu as pltpu
from jax.experimental.pallas import tpu_sc as plsc

_N = 10000
_E = 320000
_D = 128
_NC = 2
_NS = 16
_NW = _NC * _NS
_EPW = _E // _NW
_CH = 100
_IG = 10
_NG = _EPW // (_CH * _IG)
_RPT = 624
_REM = _N - _NS * _RPT
_L = 16


def _sc_mesh():
    return plsc.VectorSubcoreMesh(core_axis_name="c", subcore_axis_name="s")


def _zero_fill(buf, nrows):
    def zrow(i, _):
        for j in range(_D // _L):
            buf[i, pl.ds(j * _L, _L)] = jnp.zeros((_L,), jnp.float32)
        return 0

    lax.fori_loop(0, nrows, zrow, 0)


def _zero_acc(acc, zsrc, base):
    for k in range(6):
        pltpu.sync_copy(zsrc, acc.at[pl.ds(base + k * 96, 96)])
    pltpu.sync_copy(zsrc.at[pl.ds(0, 48)], acc.at[pl.ds(base + 576, 48)])


def _write_out(src_ref, out_hbm, cid, sid, base):
    sl = pl.ds(base, _RPT)
    pltpu.sync_copy(src_ref.at[sl], out_hbm.at[cid, sl])

    @pl.when(sid == _NS - 1)
    def _rem():
        sl2 = pl.ds(_NS * _RPT, _REM)
        pltpu.sync_copy(src_ref.at[sl2], out_hbm.at[cid, sl2])


def _make_edge_pass():
    out_type = (jax.ShapeDtypeStruct((_NC, _N, _D), jnp.float32),)
    scratch = (
        pltpu.VMEM((_IG, _CH), jnp.int32),
        pltpu.VMEM((_IG, _CH), jnp.int32),
        pltpu.VMEM((_CH, _D), jnp.float32),
        pltpu.VMEM((_CH, _D), jnp.float32),
        pltpu.VMEM_SHARED((_N, _D), jnp.float32),
        pltpu.SemaphoreType.DMA,
        pltpu.SemaphoreType.DMA,
        pltpu.SemaphoreType.DMA,
    )

    def body(y_hbm, ei_hbm, agg_hbm, src_v, dst_v, rows_a, rows_b,
             acc, semg, sema, semb):
        cid = lax.axis_index("c")
        sid = lax.axis_index("s")
        wid = cid * _NS + sid
        base = pl.multiple_of(sid * _RPT, 8)

        _zero_fill(rows_a, 96)
        _zero_acc(acc, rows_a.at[pl.ds(0, 96)], base)

        @pl.when(sid == _NS - 1)
        def _zrem():
            pltpu.sync_copy(rows_a.at[pl.ds(0, _REM)],
                            acc.at[pl.ds(_NS * _RPT, _REM)])

        plsc.subcore_barrier()

        for g in range(_NG):
            pltpu.sync_copy(ei_hbm.at[0, wid, g], src_v)
            pltpu.sync_copy(ei_hbm.at[1, wid, g], dst_v)
            pltpu.async_copy(y_hbm.at[src_v.at[0]], rows_a, semg)

            def pair(k, _):
                j = 2 * k
                pltpu.make_async_copy(y_hbm.at[src_v.at[j]], rows_a,
                                      semg).wait()
                sca = pltpu.async_copy(rows_a, acc.at[dst_v.at[j]], sema,
                                       add=True)
                pltpu.async_copy(y_hbm.at[src_v.at[j + 1]], rows_b, semg)
                pltpu.make_async_copy(y_hbm.at[src_v.at[j + 1]], rows_b,
                                      semg).wait()
                scb = pltpu.async_copy(rows_b, acc.at[dst_v.at[j + 1]], semb,
                                       add=True)
                sca.wait()

                @pl.when(k < _IG // 2 - 1)
                def _prefetch():
                    pltpu.async_copy(y_hbm.at[src_v.at[j + 2]], rows_a, semg)

                scb.wait()
                return 0

            lax.fori_loop(0, _IG // 2, pair, 0)

        plsc.subcore_barrier()
        _write_out(acc, agg_hbm, cid, sid, base)

    return pl.kernel(body, out_type=out_type, mesh=_sc_mesh(),
                     scratch_types=scratch)


def _make_cnt_pass():
    out_type = (jax.ShapeDtypeStruct((_NC, _N, _D), jnp.float32),)
    scratch = (
        pltpu.VMEM((_IG, _CH), jnp.int32),
        pltpu.VMEM((_CH, _D), jnp.float32),
        pltpu.VMEM_SHARED((_N, _D), jnp.float32),
        pltpu.SemaphoreType.DMA,
    )

    def body(ei_hbm, cnt_hbm, dst_v, ones_v, cacc, sem):
        cid = lax.axis_index("c")
        sid = lax.axis_index("s")
        wid = cid * _NS + sid
        base = pl.multiple_of(sid * _RPT, 8)

        _zero_fill(ones_v, 96)
        _zero_acc(cacc, ones_v.at[pl.ds(0, 96)], base)

        @pl.when(sid == _NS - 1)
        def _zrem():
            pltpu.sync_copy(ones_v.at[pl.ds(0, _REM)],
                            cacc.at[pl.ds(_NS * _RPT, _REM)])

        def orow(i, _):
            for j in range(_D // _L):
                ones_v[i, pl.ds(j * _L, _L)] = jnp.ones((_L,), jnp.float32)

            return 0

        lax.fori_loop(0, _CH, orow, 0)

        plsc.subcore_barrier()

        for g in range(_NG):
            pltpu.sync_copy(ei_hbm.at[1, wid, g], dst_v)

            def sc1(j, _):
                pltpu.async_copy(ones_v, cacc.at[dst_v.at[j]], sem, add=True)

                @pl.when(j >= 4)
                def _drain_one():
                    pltpu.make_async_copy(
                        ones_v, cacc.at[dst_v.at[0]], sem).wait()

                return 0

            lax.fori_loop(0, _IG, sc1, 0)
            for _ in range(4):
                pltpu.make_async_copy(ones_v, cacc.at[dst_v.at[0]],
                                      sem).wait()

        plsc.subcore_barrier()
        _write_out(cacc, cnt_hbm, cid, sid, base)

    return pl.kernel(body, out_type=out_type, mesh=_sc_mesh(),
                     scratch_types=scratch)


_edge_pass = _make_edge_pass()
_cnt_pass = _make_cnt_pass()


_BN = 2000
_GN = _N // _BN


def _dot_t(a, w):
    return lax.dot_general(a, w, (((1,), (1,)), ((), ())),
                           preferred_element_type=jnp.float32)


def _full(shape):
    return pl.BlockSpec(shape, lambda i: (0,) * len(shape))


def _rows(w=_D):
    return pl.BlockSpec((_BN, w), lambda i: (i, 0))


def _part(p):
    return pl.BlockSpec((1, _BN, _D), lambda i, _p=p: (_p, i, 0))


def _tc_pre(x, Wl, Wr, bl):
    def bdy(x_ref, wl_ref, wr_ref, bl_ref, y_ref, r_ref):
        xb = x_ref[...]
        y_ref[...] = _dot_t(xb, wl_ref[...])
        r_ref[...] = _dot_t(xb, wr_ref[...]) + bl_ref[...]

    return pl.pallas_call(
        bdy,
        grid=(_GN,),
        in_specs=[_rows(), _full((_D, _D)), _full((_D, _D)),
                  _full((1, _D))],
        out_specs=[_rows(), _rows()],
        out_shape=[jax.ShapeDtypeStruct((_N, _D), jnp.float32)] * 2,
    )(x, Wl, Wr, bl.reshape(1, _D))


def _combine(a0, a1, c0, c1, r, g, be):
    cnt = c0[:, 0:1] + c1[:, 0:1]
    inv = 1.0 / jnp.maximum(cnt, 1.0)
    t = (a0 + a1) * inv + r
    mu = jnp.mean(t, axis=-1, keepdims=True)
    var = jnp.mean((t - mu) ** 2, axis=-1, keepdims=True)
    t = (t - mu) * lax.rsqrt(var + 1e-5) * g + be
    return jnp.maximum(t, 0.0)


def _combine_refs(a0_ref, a1_ref, c0_ref, c1_ref, r_ref, g_ref, be_ref):
    return _combine(a0_ref[0], a1_ref[0], c0_ref[0], c1_ref[0],
                    r_ref[...], g_ref[...], be_ref[...])


def _tc_mid(residual):
    def bdy(a0_ref, a1_ref, c0_ref, c1_ref, r_ref, *rest):
        if residual:
            (h_ref, g_ref, be_ref, wl_ref, wr_ref, bl_ref,
             h_out, y_out, r_out) = rest
        else:
            (g_ref, be_ref, wl_ref, wr_ref, bl_ref,
             h_out, y_out, r_out) = rest
        t = _combine_refs(a0_ref, a1_ref, c0_ref, c1_ref, r_ref,
                          g_ref, be_ref)
        if residual:
            t = t + h_ref[...]
        h_out[...] = t
        y_out[...] = _dot_t(t, wl_ref[...])
        r_out[...] = _dot_t(t, wr_ref[...]) + bl_ref[...]

    n_h = [_rows()] if residual else []
    call = pl.pallas_call(
        bdy,
        grid=(_GN,),
        in_specs=([_part(0), _part(1), _part(0), _part(1), _rows()]
                  + n_h
                  + [_full((1, _D)), _full((1, _D)), _full((_D, _D)),
                     _full((_D, _D)), _full((1, _D))]),
        out_specs=[_rows()] * 3,
        out_shape=[jax.ShapeDtypeStruct((_N, _D), jnp.float32)] * 3,
    )

    def run(agg, cnt, r, h, g, be, Wl, Wr, bl):
        args = [agg, agg, cnt, cnt, r] + ([h] if residual else []) + [
            g.reshape(1, _D), be.reshape(1, _D), Wl, Wr, bl.reshape(1, _D)]
        return call(*args)

    return run


_tc_mid0 = _tc_mid(False)
_tc_mid1 = _tc_mid(True)


def _tc_final(agg, cnt, r, h, g, be, Wc1, bc1, Wc2, bc2):
    def bdy(a0_ref, a1_ref, c0_ref, c1_ref, r_ref, h_ref, g_ref, be_ref,
            wc1_ref, bc1_ref, wc2_ref, bc2_ref, out_ref):
        t = _combine_refs(a0_ref, a1_ref, c0_ref, c1_ref, r_ref,
                          g_ref, be_ref)
        t = t + h_ref[...]
        z = jnp.maximum(_dot_t(t, wc1_ref[...]) + bc1_ref[...], 0.0)
        out_ref[...] = _dot_t(z, wc2_ref[...]) + bc2_ref[0]

    Wc2p = jnp.pad(Wc2, ((0, 7), (0, 0)))

    return pl.pallas_call(
        bdy,
        grid=(_GN,),
        in_specs=[_part(0), _part(1), _part(0), _part(1), _rows(),
                  _rows(), _full((1, _D)), _full((1, _D)),
                  _full((_D // 2, _D)), _full((1, _D // 2)),
                  _full((8, _D // 2)),
                  pl.BlockSpec(memory_space=pltpu.SMEM)],
        out_specs=[pl.BlockSpec((_BN, 8), lambda i: (i, 0))],
        out_shape=[jax.ShapeDtypeStruct((_N, 8), jnp.float32)],
    )(agg, agg, cnt, cnt, r, h, g.reshape(1, _D), be.reshape(1, _D),
      Wc1, bc1.reshape(1, _D // 2), Wc2p, bc2)[0]


def kernel(x, edge_index, Wl0, bl0, Wr0, g0, be0, Wl1, bl1, Wr1, g1, be1,
           Wl2, bl2, Wr2, g2, be2, Wc1, bc1, Wc2, bc2):
    ei = edge_index.reshape(2, _NW, _NG, _IG, _CH)

    y0, r0 = _tc_pre(x, Wl0, Wr0, bl0)
    (cnt,) = _cnt_pass(ei)
    (agg0,) = _edge_pass(y0, ei)
    h1, y1, r1 = _tc_mid0(agg0, cnt, r0, None, g0, be0, Wl1, Wr1, bl1)
    (agg1,) = _edge_pass(y1, ei)
    h2, y2, r2 = _tc_mid1(agg1, cnt, r1, h1, g1, be1, Wl2, Wr2, bl2)
    (agg2,) = _edge_pass(y2, ei)
    out = _tc_final(agg2, cnt, r2, h2, g2, be2, Wc1, bc1, Wc2, bc2)
    return out[:, 0]


# speedup vs baseline: 9.4088x; 1.0685x over previous
import jax
import jax.numpy as jnp
from jax import lax
from jax.experimental import pallas as pl
from jax.experimental.pallas import tpu as pltpu
from jax.experimental.pallas import tpu_sc as plsc

_N = 10000
_E = 320000
_D = 128
_NC = 2
_NS = 16
_NW = _NC * _NS
_EPW = _E // _NW
_CH = 100
_IG = 10
_NG = _EPW // (_CH * _IG)
_RPT = 624
_REM = _N - _NS * _RPT
_L = 16


def _sc_mesh():
    return plsc.VectorSubcoreMesh(core_axis_name="c", subcore_axis_name="s")


def _zero_fill(buf, nrows):
    def zrow(i, _):
        for j in range(_D // _L):
            buf[i, pl.ds(j * _L, _L)] = jnp.zeros((_L,), jnp.float32)
        return 0

    lax.fori_loop(0, nrows, zrow, 0)


def _zero_acc(acc, zsrc, base):
    for k in range(6):
        pltpu.sync_copy(zsrc, acc.at[pl.ds(base + k * 96, 96)])
    pltpu.sync_copy(zsrc.at[pl.ds(0, 48)], acc.at[pl.ds(base + 576, 48)])


def _write_out(src_ref, out_hbm, cid, sid, base):
    sl = pl.ds(base, _RPT)
    pltpu.sync_copy(src_ref.at[sl], out_hbm.at[cid, sl])

    @pl.when(sid == _NS - 1)
    def _rem():
        sl2 = pl.ds(_NS * _RPT, _REM)
        pltpu.sync_copy(src_ref.at[sl2], out_hbm.at[cid, sl2])


def _make_edge_pass():
    out_type = (jax.ShapeDtypeStruct((_NC, _N, _D), jnp.float32),)
    scratch = (
        pltpu.VMEM((_IG, _CH), jnp.int32),
        pltpu.VMEM((_IG, _CH), jnp.int32),
        pltpu.VMEM((_CH, _D), jnp.float32),
        pltpu.VMEM((_CH, _D), jnp.float32),
        pltpu.VMEM_SHARED((_N, _D), jnp.float32),
        pltpu.SemaphoreType.DMA,
        pltpu.SemaphoreType.DMA,
        pltpu.SemaphoreType.DMA,
    )

    def body(y_hbm, ei_hbm, agg_hbm, src_v, dst_v, rows_a, rows_b,
             acc, semg, sema, semb):
        cid = lax.axis_index("c")
        sid = lax.axis_index("s")
        wid = cid * _NS + sid
        base = pl.multiple_of(sid * _RPT, 8)

        _zero_fill(rows_a, 96)
        _zero_acc(acc, rows_a.at[pl.ds(0, 96)], base)

        @pl.when(sid == _NS - 1)
        def _zrem():
            pltpu.sync_copy(rows_a.at[pl.ds(0, _REM)],
                            acc.at[pl.ds(_NS * _RPT, _REM)])

        plsc.subcore_barrier()

        for g in range(_NG):
            pltpu.sync_copy(ei_hbm.at[0, wid, g], src_v)
            pltpu.sync_copy(ei_hbm.at[1, wid, g], dst_v)
            pltpu.async_copy(y_hbm.at[src_v.at[0]], rows_a, semg)

            def pair(k, _):
                j = 2 * k
                pltpu.make_async_copy(y_hbm.at[src_v.at[j]], rows_a,
                                      semg).wait()
                sca = pltpu.async_copy(rows_a, acc.at[dst_v.at[j]], sema,
                                       add=True)
                pltpu.async_copy(y_hbm.at[src_v.at[j + 1]], rows_b, semg)
                pltpu.make_async_copy(y_hbm.at[src_v.at[j + 1]], rows_b,
                                      semg).wait()
                scb = pltpu.async_copy(rows_b, acc.at[dst_v.at[j + 1]], semb,
                                       add=True)
                sca.wait()

                @pl.when(k < _IG // 2 - 1)
                def _prefetch():
                    pltpu.async_copy(y_hbm.at[src_v.at[j + 2]], rows_a, semg)

                scb.wait()
                return 0

            lax.fori_loop(0, _IG // 2, pair, 0)

        plsc.subcore_barrier()
        _write_out(acc, agg_hbm, cid, sid, base)

    return pl.kernel(body, out_type=out_type, mesh=_sc_mesh(),
                     scratch_types=scratch)


_CR = 128
_CB = 8
_CW = _CB * _D
_NT10 = _N // _CW


def _make_cnt_pass():
    out_type = (
        jax.ShapeDtypeStruct((_NC, _NS, _CB, _D), jnp.float32),
        jax.ShapeDtypeStruct((_NC, _NS, _CR, _D), jnp.float32),
    )
    scratch = (
        pltpu.VMEM((_EPW // _L, _L), jnp.int32),
        pltpu.VMEM((_CR, _D), jnp.float32),
        pltpu.VMEM((_NS, _CB, _D), jnp.float32),
        pltpu.VMEM((_CB, _D), jnp.float32),
    )

    def body(ei_hbm, cnt_hbm, stage_hbm, idx_v, loc_v, col_v, sum_v):
        cid = lax.axis_index("c")
        sid = lax.axis_index("s")
        wid = cid * _NS + sid

        def zf(i, _):
            for j in range(_D // _L):
                loc_v[i, pl.ds(j * _L, _L)] = jnp.zeros((_L,), jnp.float32)
            return 0

        lax.fori_loop(0, _CR, zf, 0)
        pltpu.sync_copy(ei_hbm.at[1, wid], idx_v)
        ones = jnp.ones((_L,), jnp.float32)

        def step(i, _):
            idx = idx_v[i, pl.ds(0, _L)]
            plsc.addupdate_scatter(
                loc_v, [lax.shift_right_logical(idx, 7), idx & 127], ones)
            return 0

        lax.fori_loop(0, _EPW // _L, step, 0)
        pltpu.sync_copy(loc_v, stage_hbm.at[cid, sid])
        plsc.subcore_barrier()

        rbase = pl.multiple_of(sid * _CB, 8)
        pltpu.sync_copy(
            stage_hbm.at[cid, pl.ds(0, _NS), pl.ds(rbase, _CB)], col_v)

        def merge(j, _):
            r, c = j // (_D // _L), j % (_D // _L)
            sl = pl.ds(c * _L, _L)
            acc = col_v[0, r, sl]
            for t in range(1, _NS):
                acc = acc + col_v[t, r, sl]
            sum_v[r, sl] = acc
            return 0

        lax.fori_loop(0, _CW // _L, merge, 0)
        pltpu.sync_copy(sum_v, cnt_hbm.at[cid, sid])

    return pl.kernel(
        body, out_type=out_type, mesh=_sc_mesh(), scratch_types=scratch,
        compiler_params=pltpu.CompilerParams(needs_layout_passes=False))


_edge_pass = _make_edge_pass()
_cnt_pass = _make_cnt_pass()


_BN = 2000
_GN = _N // _BN


def _dot_t(a, w):
    return lax.dot_general(a, w, (((1,), (1,)), ((), ())),
                           preferred_element_type=jnp.float32)


def _full(shape):
    return pl.BlockSpec(shape, lambda i: (0,) * len(shape))


def _rows(w=_D):
    return pl.BlockSpec((_BN, w), lambda i: (i, 0))


def _part(p, w=_D):
    return pl.BlockSpec((1, _BN, w), lambda i, _p=p: (_p, i, 0))


def _tc_pre(x, Wl, Wr, bl):
    def bdy(x_ref, wl_ref, wr_ref, bl_ref, y_ref, r_ref):
        xb = x_ref[...]
        y_ref[...] = _dot_t(xb, wl_ref[...])
        r_ref[...] = _dot_t(xb, wr_ref[...]) + bl_ref[...]

    return pl.pallas_call(
        bdy,
        grid=(_GN,),
        in_specs=[_rows(), _full((_D, _D)), _full((_D, _D)),
                  _full((1, _D))],
        out_specs=[_rows(), _rows()],
        out_shape=[jax.ShapeDtypeStruct((_N, _D), jnp.float32)] * 2,
    )(x, Wl, Wr, bl.reshape(1, _D))


def _combine(a0, a1, c0, c1, r, g, be):
    cnt = c0 + c1
    inv = 1.0 / jnp.maximum(cnt, 1.0)
    t = (a0 + a1) * inv + r
    mu = jnp.mean(t, axis=-1, keepdims=True)
    var = jnp.mean((t - mu) ** 2, axis=-1, keepdims=True)
    t = (t - mu) * lax.rsqrt(var + 1e-5) * g + be
    return jnp.maximum(t, 0.0)


def _combine_refs(a0_ref, a1_ref, c0_ref, c1_ref, r_ref, g_ref, be_ref):
    return _combine(a0_ref[0], a1_ref[0], c0_ref[0], c1_ref[0],
                    r_ref[...], g_ref[...], be_ref[...])


def _tc_mid(residual):
    def bdy(a0_ref, a1_ref, c0_ref, c1_ref, r_ref, *rest):
        if residual:
            (h_ref, g_ref, be_ref, wl_ref, wr_ref, bl_ref,
             h_out, y_out, r_out) = rest
        else:
            (g_ref, be_ref, wl_ref, wr_ref, bl_ref,
             h_out, y_out, r_out) = rest
        t = _combine_refs(a0_ref, a1_ref, c0_ref, c1_ref, r_ref,
                          g_ref, be_ref)
        if residual:
            t = t + h_ref[...]
        h_out[...] = t
        y_out[...] = _dot_t(t, wl_ref[...])
        r_out[...] = _dot_t(t, wr_ref[...]) + bl_ref[...]

    n_h = [_rows()] if residual else []
    call = pl.pallas_call(
        bdy,
        grid=(_GN,),
        in_specs=([_part(0), _part(1), _part(0, 1), _part(1, 1), _rows()]
                  + n_h
                  + [_full((1, _D)), _full((1, _D)), _full((_D, _D)),
                     _full((_D, _D)), _full((1, _D))]),
        out_specs=[_rows()] * 3,
        out_shape=[jax.ShapeDtypeStruct((_N, _D), jnp.float32)] * 3,
    )

    def run(agg, cnt, r, h, g, be, Wl, Wr, bl):
        args = [agg, agg, cnt, cnt, r] + ([h] if residual else []) + [
            g.reshape(1, _D), be.reshape(1, _D), Wl, Wr, bl.reshape(1, _D)]
        return call(*args)

    return run


_tc_mid0 = _tc_mid(False)
_tc_mid1 = _tc_mid(True)


def _tc_final(agg, cnt, r, h, g, be, Wc1, bc1, Wc2, bc2):
    def bdy(a0_ref, a1_ref, c0_ref, c1_ref, r_ref, h_ref, g_ref, be_ref,
            wc1_ref, bc1_ref, wc2_ref, bc2_ref, out_ref):
        t = _combine_refs(a0_ref, a1_ref, c0_ref, c1_ref, r_ref,
                          g_ref, be_ref)
        t = t + h_ref[...]
        z = jnp.maximum(_dot_t(t, wc1_ref[...]) + bc1_ref[...], 0.0)
        out_ref[...] = _dot_t(z, wc2_ref[...]) + bc2_ref[0]

    Wc2p = jnp.pad(Wc2, ((0, 7), (0, 0)))

    return pl.pallas_call(
        bdy,
        grid=(_GN,),
        in_specs=[_part(0), _part(1), _part(0, 1), _part(1, 1), _rows(),
                  _rows(), _full((1, _D)), _full((1, _D)),
                  _full((_D // 2, _D)), _full((1, _D // 2)),
                  _full((8, _D // 2)),
                  pl.BlockSpec(memory_space=pltpu.SMEM)],
        out_specs=[pl.BlockSpec((_BN, 8), lambda i: (i, 0))],
        out_shape=[jax.ShapeDtypeStruct((_N, 8), jnp.float32)],
    )(agg, agg, cnt, cnt, r, h, g.reshape(1, _D), be.reshape(1, _D),
      Wc1, bc1.reshape(1, _D // 2), Wc2p, bc2)[0]


def kernel(x, edge_index, Wl0, bl0, Wr0, g0, be0, Wl1, bl1, Wr1, g1, be1,
           Wl2, bl2, Wr2, g2, be2, Wc1, bc1, Wc2, bc2):
    ei = edge_index.reshape(2, _NW, _NG, _IG, _CH)

    y0, r0 = _tc_pre(x, Wl0, Wr0, bl0)
    ei4 = edge_index.reshape(2, _NW, _EPW // _L, _L)
    cntm, _ = _cnt_pass(ei4)
    cnt = cntm.reshape(_NC, _NS * _CB * _D, 1)[:, :_N]
    (agg0,) = _edge_pass(y0, ei)
    h1, y1, r1 = _tc_mid0(agg0, cnt, r0, None, g0, be0, Wl1, Wr1, bl1)
    (agg1,) = _edge_pass(y1, ei)
    h2, y2, r2 = _tc_mid1(agg1, cnt, r1, h1, g1, be1, Wl2, Wr2, bl2)
    (agg2,) = _edge_pass(y2, ei)
    out = _tc_final(agg2, cnt, r2, h2, g2, be2, Wc1, bc1, Wc2, bc2)
    return out[:, 0]


# speedup vs baseline: 9.5565x; 1.0157x over previous
import jax
import jax.numpy as jnp
from jax import lax
from jax.experimental import pallas as pl
from jax.experimental.pallas import tpu as pltpu
from jax.experimental.pallas import tpu_sc as plsc

_N = 10000
_E = 320000
_D = 128
_NC = 2
_NS = 16
_NW = _NC * _NS
_EPW = _E // _NW
_CH = 100
_IG = 10
_NG = _EPW // (_CH * _IG)
_RPT = 624
_REM = _N - _NS * _RPT
_L = 16


def _sc_mesh():
    return plsc.VectorSubcoreMesh(core_axis_name="c", subcore_axis_name="s")


def _zero_fill(buf, nrows):
    def zrow(i, _):
        for j in range(_D // _L):
            buf[i, pl.ds(j * _L, _L)] = jnp.zeros((_L,), jnp.float32)
        return 0

    lax.fori_loop(0, nrows, zrow, 0)


def _zero_acc(acc, zsrc, base):
    for k in range(6):
        pltpu.sync_copy(zsrc, acc.at[pl.ds(base + k * 96, 96)])
    pltpu.sync_copy(zsrc.at[pl.ds(0, 48)], acc.at[pl.ds(base + 576, 48)])


def _write_out(src_ref, out_hbm, cid, sid, base):
    sl = pl.ds(base, _RPT)
    pltpu.sync_copy(src_ref.at[sl], out_hbm.at[cid, sl])

    @pl.when(sid == _NS - 1)
    def _rem():
        sl2 = pl.ds(_NS * _RPT, _REM)
        pltpu.sync_copy(src_ref.at[sl2], out_hbm.at[cid, sl2])


def _make_edge_pass():
    out_type = (jax.ShapeDtypeStruct((_NC, _N, _D), jnp.float32),)
    scratch = (
        pltpu.VMEM((_IG, _CH), jnp.int32),
        pltpu.VMEM((_IG, _CH), jnp.int32),
        pltpu.VMEM((_CH, _D), jnp.float32),
        pltpu.VMEM((_CH, _D), jnp.float32),
        pltpu.VMEM_SHARED((_N, _D), jnp.float32),
        pltpu.SemaphoreType.DMA,
        pltpu.SemaphoreType.DMA,
        pltpu.SemaphoreType.DMA,
    )

    def body(y_hbm, ei_hbm, agg_hbm, src_v, dst_v, rows_a, rows_b,
             acc, semg, sema, semb):
        cid = lax.axis_index("c")
        sid = lax.axis_index("s")
        wid = cid * _NS + sid
        base = pl.multiple_of(sid * _RPT, 8)

        _zero_fill(rows_a, 96)
        _zero_acc(acc, rows_a.at[pl.ds(0, 96)], base)

        @pl.when(sid == _NS - 1)
        def _zrem():
            pltpu.sync_copy(rows_a.at[pl.ds(0, _REM)],
                            acc.at[pl.ds(_NS * _RPT, _REM)])

        plsc.subcore_barrier()

        for g in range(_NG):
            pltpu.sync_copy(ei_hbm.at[0, wid, g], src_v)
            pltpu.sync_copy(ei_hbm.at[1, wid, g], dst_v)
            pltpu.async_copy(y_hbm.at[src_v.at[0]], rows_a, semg)

            def pair(k, _):
                j = 2 * k
                pltpu.make_async_copy(y_hbm.at[src_v.at[j]], rows_a,
                                      semg).wait()
                sca = pltpu.async_copy(rows_a, acc.at[dst_v.at[j]], sema,
                                       add=True)
                pltpu.async_copy(y_hbm.at[src_v.at[j + 1]], rows_b, semg)
                pltpu.make_async_copy(y_hbm.at[src_v.at[j + 1]], rows_b,
                                      semg).wait()
                scb = pltpu.async_copy(rows_b, acc.at[dst_v.at[j + 1]], semb,
                                       add=True)
                sca.wait()

                @pl.when(k < _IG // 2 - 1)
                def _prefetch():
                    pltpu.async_copy(y_hbm.at[src_v.at[j + 2]], rows_a, semg)

                scb.wait()
                return 0

            lax.fori_loop(0, _IG // 2, pair, 0)

        plsc.subcore_barrier()
        _write_out(acc, agg_hbm, cid, sid, base)

    return pl.kernel(body, out_type=out_type, mesh=_sc_mesh(),
                     scratch_types=scratch)


_CR = 128
_CB = 8
_CW = _CB * _D
_NT10 = _N // _CW


def _make_cnt_pass():
    out_type = (
        jax.ShapeDtypeStruct((_NC, _NS, _CB, _D), jnp.float32),
        jax.ShapeDtypeStruct((_NC, _NS, _CR, _D), jnp.float32),
    )
    scratch = (
        pltpu.VMEM((_EPW // _L, _L), jnp.int32),
        pltpu.VMEM((_CR, _D), jnp.float32),
        pltpu.VMEM((_NS, _CB, _D), jnp.float32),
        pltpu.VMEM((_CB, _D), jnp.float32),
    )

    def body(ei_hbm, cnt_hbm, stage_hbm, idx_v, loc_v, col_v, sum_v):
        cid = lax.axis_index("c")
        sid = lax.axis_index("s")
        wid = cid * _NS + sid

        def zf(i, _):
            for j in range(_D // _L):
                loc_v[i, pl.ds(j * _L, _L)] = jnp.zeros((_L,), jnp.float32)
            return 0

        lax.fori_loop(0, _CR, zf, 0)
        pltpu.sync_copy(ei_hbm.at[1, wid], idx_v)
        ones = jnp.ones((_L,), jnp.float32)

        def step(i, _):
            idx = idx_v[i, pl.ds(0, _L)]
            plsc.addupdate_scatter(
                loc_v, [lax.shift_right_logical(idx, 7), idx & 127], ones)
            return 0

        lax.fori_loop(0, _EPW // _L, step, 0)
        pltpu.sync_copy(loc_v, stage_hbm.at[cid, sid])
        plsc.subcore_barrier()

        rbase = pl.multiple_of(sid * _CB, 8)
        pltpu.sync_copy(
            stage_hbm.at[cid, pl.ds(0, _NS), pl.ds(rbase, _CB)], col_v)

        def merge(j, _):
            r, c = j // (_D // _L), j % (_D // _L)
            sl = pl.ds(c * _L, _L)
            acc = col_v[0, r, sl]
            for t in range(1, _NS):
                acc = acc + col_v[t, r, sl]
            sum_v[r, sl] = acc
            return 0

        lax.fori_loop(0, _CW // _L, merge, 0)
        pltpu.sync_copy(sum_v, cnt_hbm.at[cid, sid])

    return pl.kernel(
        body, out_type=out_type, mesh=_sc_mesh(), scratch_types=scratch,
        compiler_params=pltpu.CompilerParams(needs_layout_passes=False))


_edge_pass = _make_edge_pass()
_cnt_pass = _make_cnt_pass()


_BN = 2000
_GN = _N // _BN


def _dot_t(a, w):
    return lax.dot_general(a, w, (((1,), (1,)), ((), ())),
                           preferred_element_type=jnp.float32)


def _full(shape):
    return pl.BlockSpec(shape, lambda i: (0,) * len(shape))


def _rows(w=_D):
    return pl.BlockSpec((_BN, w), lambda i: (i, 0))


def _part(p, w=_D):
    return pl.BlockSpec((1, _BN, w), lambda i, _p=p: (_p, i, 0))


def _tc_mid(residual):
    def bdy(a0_ref, a1_ref, c0_ref, c1_ref, h_ref, g_ref, be_ref, wl_ref,
            wr_ref, bl_ref, h_out):
        cnt = c0_ref[0] + c1_ref[0]
        inv = 1.0 / jnp.maximum(cnt, 1.0)
        agg = (a0_ref[0] + a1_ref[0]) * inv
        hb = h_ref[...]
        t = (_dot_t(agg, wl_ref[...]) + bl_ref[...]
             + _dot_t(hb, wr_ref[...]))
        mu = jnp.mean(t, axis=-1, keepdims=True)
        var = jnp.mean((t - mu) ** 2, axis=-1, keepdims=True)
        t = (t - mu) * lax.rsqrt(var + 1e-5) * g_ref[...] + be_ref[...]
        t = jnp.maximum(t, 0.0)
        if residual:
            t = t + hb
        h_out[...] = t

    call = pl.pallas_call(
        bdy,
        grid=(_GN,),
        in_specs=[_part(0), _part(1), _part(0, 1), _part(1, 1), _rows(),
                  _full((1, _D)), _full((1, _D)), _full((_D, _D)),
                  _full((_D, _D)), _full((1, _D))],
        out_specs=[_rows()],
        out_shape=[jax.ShapeDtypeStruct((_N, _D), jnp.float32)],
    )

    def run(agg, cnt, h, g, be, Wl, Wr, bl):
        return call(agg, agg, cnt, cnt, h, g.reshape(1, _D),
                    be.reshape(1, _D), Wl, Wr, bl.reshape(1, _D))[0]

    return run


_tc_mid0 = _tc_mid(False)
_tc_mid1 = _tc_mid(True)


def _tc_final(agg, cnt, h, g, be, Wl, Wr, bl, Wc1, bc1, Wc2, bc2):
    def bdy(a0_ref, a1_ref, c0_ref, c1_ref, h_ref, g_ref, be_ref, wl_ref,
            wr_ref, bl_ref, wc1_ref, bc1_ref, wc2_ref, bc2_ref, out_ref):
        cnt2 = c0_ref[0] + c1_ref[0]
        inv = 1.0 / jnp.maximum(cnt2, 1.0)
        agg2 = (a0_ref[0] + a1_ref[0]) * inv
        hb = h_ref[...]
        t = (_dot_t(agg2, wl_ref[...]) + bl_ref[...]
             + _dot_t(hb, wr_ref[...]))
        mu = jnp.mean(t, axis=-1, keepdims=True)
        var = jnp.mean((t - mu) ** 2, axis=-1, keepdims=True)
        t = (t - mu) * lax.rsqrt(var + 1e-5) * g_ref[...] + be_ref[...]
        t = jnp.maximum(t, 0.0) + hb
        z = jnp.maximum(_dot_t(t, wc1_ref[...]) + bc1_ref[...], 0.0)
        out_ref[...] = _dot_t(z, wc2_ref[...]) + bc2_ref[0]

    Wc2p = jnp.pad(Wc2, ((0, 7), (0, 0)))

    return pl.pallas_call(
        bdy,
        grid=(_GN,),
        in_specs=[_part(0), _part(1), _part(0, 1), _part(1, 1), _rows(),
                  _full((1, _D)), _full((1, _D)), _full((_D, _D)),
                  _full((_D, _D)), _full((1, _D)),
                  _full((_D // 2, _D)), _full((1, _D // 2)),
                  _full((8, _D // 2)),
                  pl.BlockSpec(memory_space=pltpu.SMEM)],
        out_specs=[pl.BlockSpec((_BN, 8), lambda i: (i, 0))],
        out_shape=[jax.ShapeDtypeStruct((_N, 8), jnp.float32)],
    )(agg, agg, cnt, cnt, h, g.reshape(1, _D), be.reshape(1, _D), Wl, Wr,
      bl.reshape(1, _D), Wc1, bc1.reshape(1, _D // 2), Wc2p, bc2)[0]


def kernel(x, edge_index, Wl0, bl0, Wr0, g0, be0, Wl1, bl1, Wr1, g1, be1,
           Wl2, bl2, Wr2, g2, be2, Wc1, bc1, Wc2, bc2):
    ei = edge_index.reshape(2, _NW, _NG, _IG, _CH)
    ei4 = edge_index.reshape(2, _NW, _EPW // _L, _L)

    cntm, _ = _cnt_pass(ei4)
    cnt = cntm.reshape(_NC, _NS * _CB * _D, 1)[:, :_N]
    (agg0,) = _edge_pass(x, ei)
    h1 = _tc_mid0(agg0, cnt, x, g0, be0, Wl0, Wr0, bl0)
    (agg1,) = _edge_pass(h1, ei)
    h2 = _tc_mid1(agg1, cnt, h1, g1, be1, Wl1, Wr1, bl1)
    (agg2,) = _edge_pass(h2, ei)
    out = _tc_final(agg2, cnt, h2, g2, be2, Wl2, Wr2, bl2,
                    Wc1, bc1, Wc2, bc2)
    return out[:, 0]


# speedup vs baseline: 10.2838x; 1.0761x over previous
import jax
import jax.numpy as jnp
from jax import lax
from jax.experimental import pallas as pl
from jax.experimental.pallas import tpu as pltpu
from jax.experimental.pallas import tpu_sc as plsc

_N = 10000
_E = 320000
_D = 128
_NC = 2
_NS = 16
_NW = _NC * _NS
_EPW = _E // _NW
_CH = 100
_IG = 20
_NG = _EPW // (_CH * _IG)
_RPT = 624
_REM = _N - _NS * _RPT
_L = 16


def _sc_mesh():
    return plsc.VectorSubcoreMesh(core_axis_name="c", subcore_axis_name="s")


def _zero_fill(buf, nrows):
    def zrow(i, _):
        for j in range(_D // _L):
            buf[i, pl.ds(j * _L, _L)] = jnp.zeros((_L,), jnp.float32)
        return 0

    lax.fori_loop(0, nrows, zrow, 0)


def _zero_acc(acc, zsrc, base):
    for k in range(6):
        pltpu.sync_copy(zsrc, acc.at[pl.ds(base + k * 96, 96)])
    pltpu.sync_copy(zsrc.at[pl.ds(0, 48)], acc.at[pl.ds(base + 576, 48)])


def _write_out(src_ref, out_hbm, cid, sid, base):
    sl = pl.ds(base, _RPT)
    pltpu.sync_copy(src_ref.at[sl], out_hbm.at[cid, sl])

    @pl.when(sid == _NS - 1)
    def _rem():
        sl2 = pl.ds(_NS * _RPT, _REM)
        pltpu.sync_copy(src_ref.at[sl2], out_hbm.at[cid, sl2])


def _make_edge_pass():
    out_type = (jax.ShapeDtypeStruct((_NC, _N, _D), jnp.float32),)
    scratch = (
        pltpu.VMEM((_IG, _CH), jnp.int32),
        pltpu.VMEM((_IG, _CH), jnp.int32),
        pltpu.VMEM((_CH, _D), jnp.float32),
        pltpu.VMEM((_CH, _D), jnp.float32),
        pltpu.VMEM_SHARED((_N, _D), jnp.float32),
        pltpu.SemaphoreType.DMA,
        pltpu.SemaphoreType.DMA,
        pltpu.SemaphoreType.DMA,
    )

    def body(y_hbm, ei_hbm, agg_hbm, src_v, dst_v, rows_a, rows_b,
             acc, semg, sema, semb):
        cid = lax.axis_index("c")
        sid = lax.axis_index("s")
        wid = cid * _NS + sid
        base = pl.multiple_of(sid * _RPT, 8)

        _zero_fill(rows_a, 96)
        _zero_acc(acc, rows_a.at[pl.ds(0, 96)], base)

        @pl.when(sid == _NS - 1)
        def _zrem():
            pltpu.sync_copy(rows_a.at[pl.ds(0, _REM)],
                            acc.at[pl.ds(_NS * _RPT, _REM)])

        plsc.subcore_barrier()

        for g in range(_NG):
            pltpu.sync_copy(ei_hbm.at[0, wid, g], src_v)
            pltpu.sync_copy(ei_hbm.at[1, wid, g], dst_v)
            pltpu.async_copy(y_hbm.at[src_v.at[0]], rows_a, semg)

            def pair(k, _):
                j = 2 * k
                pltpu.make_async_copy(y_hbm.at[src_v.at[j]], rows_a,
                                      semg).wait()
                sca = pltpu.async_copy(rows_a, acc.at[dst_v.at[j]], sema,
                                       add=True)
                pltpu.async_copy(y_hbm.at[src_v.at[j + 1]], rows_b, semg)
                pltpu.make_async_copy(y_hbm.at[src_v.at[j + 1]], rows_b,
                                      semg).wait()
                scb = pltpu.async_copy(rows_b, acc.at[dst_v.at[j + 1]], semb,
                                       add=True)
                sca.wait()

                @pl.when(k < _IG // 2 - 1)
                def _prefetch():
                    pltpu.async_copy(y_hbm.at[src_v.at[j + 2]], rows_a, semg)

                scb.wait()
                return 0

            lax.fori_loop(0, _IG // 2, pair, 0)

        plsc.subcore_barrier()
        _write_out(acc, agg_hbm, cid, sid, base)

    return pl.kernel(body, out_type=out_type, mesh=_sc_mesh(),
                     scratch_types=scratch)


_CR = 128
_CB = 8
_CW = _CB * _D
_NT10 = _N // _CW


def _make_cnt_pass():
    out_type = (
        jax.ShapeDtypeStruct((_NC, _NS, _CB, _D), jnp.float32),
        jax.ShapeDtypeStruct((_NC, _NS, _CR, _D), jnp.float32),
    )
    scratch = (
        pltpu.VMEM((_IG, _CH), jnp.int32),
        pltpu.VMEM((_CR, _D), jnp.float32),
        pltpu.VMEM((_NS, _CB, _D), jnp.float32),
        pltpu.VMEM((_CB, _D), jnp.float32),
    )

    def body(ei_hbm, cnt_hbm, stage_hbm, idx_v, loc_v, col_v, sum_v):
        cid = lax.axis_index("c")
        sid = lax.axis_index("s")
        wid = cid * _NS + sid

        def zf(i, _):
            for j in range(_D // _L):
                loc_v[i, pl.ds(j * _L, _L)] = jnp.zeros((_L,), jnp.float32)
            return 0

        lax.fori_loop(0, _CR, zf, 0)
        ones = jnp.ones((_L,), jnp.float32)
        tail_mask = lax.iota(jnp.int32, _L) >= (_L - (_CH - (_CH // _L) * _L))
        nfull = _CH // _L
        tail_off = _CH - _L

        def count(idx, mask=None):
            plsc.addupdate_scatter(
                loc_v, [lax.shift_right_logical(idx, 7), idx & 127], ones,
                mask=mask)

        for g in range(_NG):
            pltpu.sync_copy(ei_hbm.at[1, wid, g], idx_v)

            def step(i, _):
                for k in range(nfull):
                    count(idx_v[i, pl.ds(k * _L, _L)])
                count(idx_v[i, pl.ds(tail_off, _L)], tail_mask)
                return 0

            lax.fori_loop(0, _IG, step, 0)
        pltpu.sync_copy(loc_v, stage_hbm.at[cid, sid])
        plsc.subcore_barrier()

        rbase = pl.multiple_of(sid * _CB, 8)
        pltpu.sync_copy(
            stage_hbm.at[cid, pl.ds(0, _NS), pl.ds(rbase, _CB)], col_v)

        def merge(j, _):
            r, c = j // (_D // _L), j % (_D // _L)
            sl = pl.ds(c * _L, _L)
            acc = col_v[0, r, sl]
            for t in range(1, _NS):
                acc = acc + col_v[t, r, sl]
            sum_v[r, sl] = acc
            return 0

        lax.fori_loop(0, _CW // _L, merge, 0)
        pltpu.sync_copy(sum_v, cnt_hbm.at[cid, sid])

    return pl.kernel(
        body, out_type=out_type, mesh=_sc_mesh(), scratch_types=scratch,
        compiler_params=pltpu.CompilerParams(needs_layout_passes=False))


_edge_pass = _make_edge_pass()
_cnt_pass = _make_cnt_pass()


_BN = 2000
_GN = _N // _BN


def _dot_t(a, w):
    return lax.dot_general(a, w, (((1,), (1,)), ((), ())),
                           preferred_element_type=jnp.float32)


def _full(shape):
    return pl.BlockSpec(shape, lambda i: (0,) * len(shape))


def _rows(w=_D):
    return pl.BlockSpec((_BN, w), lambda i: (i, 0))


def _part(p, w=_D):
    return pl.BlockSpec((1, _BN, w), lambda i, _p=p: (_p, i, 0))


def _tc_mid(residual):
    def bdy(a0_ref, a1_ref, c0_ref, c1_ref, h_ref, g_ref, be_ref, wl_ref,
            wr_ref, bl_ref, h_out):
        cnt = c0_ref[0] + c1_ref[0]
        inv = 1.0 / jnp.maximum(cnt, 1.0)
        agg = (a0_ref[0] + a1_ref[0]) * inv
        hb = h_ref[...]
        t = (_dot_t(agg, wl_ref[...]) + bl_ref[...]
             + _dot_t(hb, wr_ref[...]))
        mu = jnp.mean(t, axis=-1, keepdims=True)
        var = jnp.mean((t - mu) ** 2, axis=-1, keepdims=True)
        t = (t - mu) * lax.rsqrt(var + 1e-5) * g_ref[...] + be_ref[...]
        t = jnp.maximum(t, 0.0)
        if residual:
            t = t + hb
        h_out[...] = t

    call = pl.pallas_call(
        bdy,
        grid=(_GN,),
        in_specs=[_part(0), _part(1), _part(0, 1), _part(1, 1), _rows(),
                  _full((1, _D)), _full((1, _D)), _full((_D, _D)),
                  _full((_D, _D)), _full((1, _D))],
        out_specs=[_rows()],
        out_shape=[jax.ShapeDtypeStruct((_N, _D), jnp.float32)],
    )

    def run(agg, cnt, h, g, be, Wl, Wr, bl):
        return call(agg, agg, cnt, cnt, h, g.reshape(1, _D),
                    be.reshape(1, _D), Wl, Wr, bl.reshape(1, _D))[0]

    return run


_tc_mid0 = _tc_mid(False)
_tc_mid1 = _tc_mid(True)


def _tc_final(agg, cnt, h, g, be, Wl, Wr, bl, Wc1, bc1, Wc2, bc2):
    def bdy(a0_ref, a1_ref, c0_ref, c1_ref, h_ref, g_ref, be_ref, wl_ref,
            wr_ref, bl_ref, wc1_ref, bc1_ref, wc2_ref, bc2_ref, out_ref):
        cnt2 = c0_ref[0] + c1_ref[0]
        inv = 1.0 / jnp.maximum(cnt2, 1.0)
        agg2 = (a0_ref[0] + a1_ref[0]) * inv
        hb = h_ref[...]
        t = (_dot_t(agg2, wl_ref[...]) + bl_ref[...]
             + _dot_t(hb, wr_ref[...]))
        mu = jnp.mean(t, axis=-1, keepdims=True)
        var = jnp.mean((t - mu) ** 2, axis=-1, keepdims=True)
        t = (t - mu) * lax.rsqrt(var + 1e-5) * g_ref[...] + be_ref[...]
        t = jnp.maximum(t, 0.0) + hb
        z = jnp.maximum(_dot_t(t, wc1_ref[...]) + bc1_ref[...], 0.0)
        out_ref[...] = _dot_t(z, wc2_ref[...]) + bc2_ref[0]

    Wc2p = jnp.pad(Wc2, ((0, 7), (0, 0)))

    return pl.pallas_call(
        bdy,
        grid=(_GN,),
        in_specs=[_part(0), _part(1), _part(0, 1), _part(1, 1), _rows(),
                  _full((1, _D)), _full((1, _D)), _full((_D, _D)),
                  _full((_D, _D)), _full((1, _D)),
                  _full((_D // 2, _D)), _full((1, _D // 2)),
                  _full((8, _D // 2)),
                  pl.BlockSpec(memory_space=pltpu.SMEM)],
        out_specs=[pl.BlockSpec((_BN, 8), lambda i: (i, 0))],
        out_shape=[jax.ShapeDtypeStruct((_N, 8), jnp.float32)],
    )(agg, agg, cnt, cnt, h, g.reshape(1, _D), be.reshape(1, _D), Wl, Wr,
      bl.reshape(1, _D), Wc1, bc1.reshape(1, _D // 2), Wc2p, bc2)[0]


def kernel(x, edge_index, Wl0, bl0, Wr0, g0, be0, Wl1, bl1, Wr1, g1, be1,
           Wl2, bl2, Wr2, g2, be2, Wc1, bc1, Wc2, bc2):
    ei = edge_index.reshape(2, _NW, _NG, _IG, _CH)

    cntm, _ = _cnt_pass(ei)
    cnt = cntm.reshape(_NC, _NS * _CB * _D, 1)
    (agg0,) = _edge_pass(x, ei)
    h1 = _tc_mid0(agg0, cnt, x, g0, be0, Wl0, Wr0, bl0)
    (agg1,) = _edge_pass(h1, ei)
    h2 = _tc_mid1(agg1, cnt, h1, g1, be1, Wl1, Wr1, bl1)
    (agg2,) = _edge_pass(h2, ei)
    out = _tc_final(agg2, cnt, h2, g2, be2, Wl2, Wr2, bl2,
                    Wc1, bc1, Wc2, bc2)
    return out[:, 0]
